# baseline (speedup 1.0000x reference)
import functools

import jax
import jax.numpy as jnp
from jax import lax
from jax.experimental import pallas as pl
from jax.experimental.pallas import tpu as pltpu
from jax.experimental.pallas import tpu_sc as plsc

B, P, H, W = 16, 4, 64, 128
N = B * P * H * W
D = 256
NC, NS = 2, 16
NW = NC * NS
NT = N // NW
C = 32
IB = 512
CB = IB // C
HB = CB // 2
NBLK = NT // IB
ST = 257
T1W = 300 * ST
T2W = 80 * ST


def _prep_body(c_ref, n_ref, l_ref, num_ref, p_ref, w_ref, dk_ref, b_ref,
               t1_ref, t2_ref):
    dk = dk_ref[...]
    t1_ref[...] = jnp.dot(c_ref[...], dk[0:64, :],
                          preferred_element_type=jnp.float32)
    n_proj = jnp.dot(n_ref[...], dk[64:80, :],
                     preferred_element_type=jnp.float32)
    l_proj = jnp.dot(l_ref[...], dk[80:112, :],
                     preferred_element_type=jnp.float32)
    num_proj = jnp.dot(num_ref[...], dk[112:144, :],
                       preferred_element_type=jnp.float32)
    p_proj = jnp.dot(p_ref[...], dk[144:176, :],
                     preferred_element_type=jnp.float32)
    w_proj = jnp.dot(w_ref[...], dk[176:208, :],
                     preferred_element_type=jnp.float32)

    idx = lax.broadcasted_iota(jnp.int32, (80, 1), 0)
    nb = idx // 16
    lbit = (idx // 8) % 2
    nbit = (idx // 4) % 2
    pbit = (idx // 2) % 2
    wbit = idx % 2

    acc = b_ref[...]
    for k in range(5):
        acc = acc + jnp.where(nb == k, 1.0, 0.0) * n_proj[k:k + 1, :]
    acc = acc + jnp.where(lbit == 1, l_proj[1:2, :], l_proj[0:1, :])
    acc = acc + jnp.where(nbit == 1, num_proj[1:2, :], num_proj[0:1, :])
    acc = acc + jnp.where(pbit == 1, p_proj[1:2, :], p_proj[0:1, :])
    acc = acc + jnp.where(wbit == 1, w_proj[1:2, :], w_proj[0:1, :])
    t2_ref[...] = acc


def _prep_tables(c_table, n_table, l_table, num_table, p_table, w_table,
                 dense_kernel, dense_bias):
    return pl.pallas_call(
        _prep_body,
        out_shape=[
            jax.ShapeDtypeStruct((300, D), jnp.float32),
            jax.ShapeDtypeStruct((80, D), jnp.float32),
        ],
    )(c_table, n_table, l_table, num_table, p_table, w_table,
      dense_kernel, dense_bias.reshape(1, D))


def _sc_body(cc, nb, il, inum, ip, iw, t1, t2, out,
             t1v, t2v,
             cc_v, nb_v, il_v, in_v, ip_v, iw_v, i1_b, i2_b,
             oS0, oS1,
             semIdx, sO0, sO1):
    wid = lax.axis_index("s") * NC + lax.axis_index("c")
    base0 = wid * NT
    ost = ((oS0, sO0), (oS1, sO1))

    pltpu.sync_copy(t1, t1v)
    pltpu.sync_copy(t2, t2v)

    def chunk(cl, blk, set_i):
        ob, osem = ost[set_i]
        gc = blk * CB + cl

        @pl.when(gc >= 2)
        def _():
            pltpu.make_async_copy(ob.at[:, pl.ds(0, D)],
                                  out.at[pl.ds(0, C)], osem).wait()

        for grp in range(C // 16):
            goff = cl * C + grp * 16
            i1s_v = i1_b[pl.ds(goff, 16)]
            i2s_v = i2_b[pl.ds(goff, 16)]
            tok_ids = lax.broadcasted_iota(jnp.int32, (16,), 0) + grp * 16

            def jbody(jb, carry2):
                for u in range(8):
                    j = jb * 8 + u
                    jv = jnp.zeros((16,), jnp.int32) + j
                    a = plsc.load_gather(t1v, [i1s_v, jv])
                    b = plsc.load_gather(t2v, [i2s_v, jv])
                    plsc.store_scatter(ob, [tok_ids, jv], a + b)
                return carry2

            lax.fori_loop(0, D // 8, jbody, 0)

        base = base0 + blk * IB + cl * C
        pltpu.async_copy(ob.at[:, pl.ds(0, D)], out.at[pl.ds(base, C)], osem)

    def block(blk, carry):
        bbase = base0 + blk * IB
        cps = [pltpu.async_copy(src.at[pl.ds(bbase, IB)], dst, semIdx)
               for src, dst in zip((cc, nb, il, inum, ip, iw),
                                   (cc_v, nb_v, il_v, in_v, ip_v, iw_v))]
        for cp in cps:
            cp.wait()

        def ixbody(j, carry2):
            sl = pl.ds(j * 16, 16)
            i1_b[sl] = lax.rem(cc_v[sl], 300)
            i2_b[sl] = nb_v[sl] * 16 + il_v[sl] * 8 + in_v[sl] * 4 \
                + ip_v[sl] * 2 + iw_v[sl]
            return carry2

        lax.fori_loop(0, IB // 16, ixbody, 0)

        def pair(h, carry2):
            c0 = 2 * h
            chunk(c0, blk, 0)
            chunk(c0 + 1, blk, 1)
            return carry2

        lax.fori_loop(0, HB, pair, 0)
        return carry

    lax.fori_loop(0, NBLK, block, 0)
    pltpu.make_async_copy(oS0.at[:, pl.ds(0, D)], out.at[pl.ds(0, C)], sO0).wait()
    pltpu.make_async_copy(oS1.at[:, pl.ds(0, D)], out.at[pl.ds(0, C)], sO1).wait()


_sc_kernel = functools.partial(
    pl.kernel,
    mesh=plsc.VectorSubcoreMesh(core_axis_name="c", subcore_axis_name="s"),
    out_type=jax.ShapeDtypeStruct((N, D), jnp.float32),
    compiler_params=pltpu.CompilerParams(needs_layout_passes=False,
                                         use_tc_tiling_on_sc=False),
    scratch_types=[
        pltpu.VMEM((300, ST), jnp.float32),
        pltpu.VMEM((80, ST), jnp.float32),
        pltpu.VMEM((IB,), jnp.int32),
        pltpu.VMEM((IB,), jnp.int32),
        pltpu.VMEM((IB,), jnp.int32),
        pltpu.VMEM((IB,), jnp.int32),
        pltpu.VMEM((IB,), jnp.int32),
        pltpu.VMEM((IB,), jnp.int32),
        pltpu.VMEM((IB,), jnp.int32),
        pltpu.VMEM((IB,), jnp.int32),
        pltpu.VMEM((C, ST), jnp.float32),
        pltpu.VMEM((C, ST), jnp.float32),
        pltpu.SemaphoreType.DMA,
        pltpu.SemaphoreType.DMA,
        pltpu.SemaphoreType.DMA,
    ],
)(_sc_body)


def kernel(char_code, num_bytes, is_letter, is_number, is_punctuation,
           is_whitespace, c_table, n_table, l_table, num_table, p_table,
           w_table, dense_kernel, dense_bias):
    t1, t2 = _prep_tables(c_table, n_table, l_table, num_table, p_table,
                          w_table, dense_kernel, dense_bias)
    t1 = jnp.pad(t1, ((0, 0), (0, ST - D)))
    t2 = jnp.pad(t2, ((0, 0), (0, ST - D)))
    cc = char_code.reshape(N).astype(jnp.int32)
    nb = num_bytes.reshape(N).astype(jnp.int32)
    il = is_letter.reshape(N).astype(jnp.int32)
    inum = is_number.reshape(N).astype(jnp.int32)
    ip = is_punctuation.reshape(N).astype(jnp.int32)
    iw = is_whitespace.reshape(N).astype(jnp.int32)
    out = _sc_kernel(cc, nb, il, inum, ip, iw, t1, t2)
    return out.reshape(B, P, H, W, D)


# speedup vs baseline: 2.1411x; 2.1411x over previous
import functools

import jax
import jax.numpy as jnp
from jax import lax
from jax.experimental import pallas as pl
from jax.experimental.pallas import tpu as pltpu
from jax.experimental.pallas import tpu_sc as plsc

B, P, H, W = 16, 4, 64, 128
N = B * P * H * W
D = 256
NC, NS = 2, 16
NW = NC * NS
NT = N // NW
C = 64
IB = 1024
CB = IB // C
HB = CB // 2
NBLK = NT // IB


def _prep_body(c_ref, n_ref, l_ref, num_ref, p_ref, w_ref, dk_ref, b_ref,
               t1_ref, t2_ref):
    dk = dk_ref[...]
    t1_ref[...] = jnp.dot(c_ref[...], dk[0:64, :],
                          preferred_element_type=jnp.float32)
    n_proj = jnp.dot(n_ref[...], dk[64:80, :],
                     preferred_element_type=jnp.float32)
    l_proj = jnp.dot(l_ref[...], dk[80:112, :],
                     preferred_element_type=jnp.float32)
    num_proj = jnp.dot(num_ref[...], dk[112:144, :],
                       preferred_element_type=jnp.float32)
    p_proj = jnp.dot(p_ref[...], dk[144:176, :],
                     preferred_element_type=jnp.float32)
    w_proj = jnp.dot(w_ref[...], dk[176:208, :],
                     preferred_element_type=jnp.float32)

    idx = lax.broadcasted_iota(jnp.int32, (80, 1), 0)
    nb = idx // 16
    lbit = (idx // 8) % 2
    nbit = (idx // 4) % 2
    pbit = (idx // 2) % 2
    wbit = idx % 2

    acc = b_ref[...]
    for k in range(5):
        acc = acc + jnp.where(nb == k, 1.0, 0.0) * n_proj[k:k + 1, :]
    acc = acc + jnp.where(lbit == 1, l_proj[1:2, :], l_proj[0:1, :])
    acc = acc + jnp.where(nbit == 1, num_proj[1:2, :], num_proj[0:1, :])
    acc = acc + jnp.where(pbit == 1, p_proj[1:2, :], p_proj[0:1, :])
    acc = acc + jnp.where(wbit == 1, w_proj[1:2, :], w_proj[0:1, :])
    t2_ref[...] = acc


def _prep_tables(c_table, n_table, l_table, num_table, p_table, w_table,
                 dense_kernel, dense_bias):
    return pl.pallas_call(
        _prep_body,
        out_shape=[
            jax.ShapeDtypeStruct((300, D), jnp.float32),
            jax.ShapeDtypeStruct((80, D), jnp.float32),
        ],
    )(c_table, n_table, l_table, num_table, p_table, w_table,
      dense_kernel, dense_bias.reshape(1, D))


def _sc_body(cc, nb, il, inum, ip, iw, t1, t2, out,
             cc_v, nb_v, il_v, in_v, ip_v, iw_v, i1_b, i2_b,
             rA1, rA2, rB1, rB2, oS0, oS1,
             semIdx, sA1, sA2, sB1, sB2, sO0, sO1):
    wid = lax.axis_index("s") * NC + lax.axis_index("c")
    base0 = wid * NT
    rows = ((rA1, rA2, sA1, sA2), (rB1, rB2, sB1, sB2))
    ost = ((oS0, sO0), (oS1, sO1))

    def issue_gather(cl, set_i):
        r1, r2, s1, s2 = rows[set_i]
        off = cl * C
        pltpu.async_copy(t1.at[i1_b.at[pl.ds(off, C)]], r1, s1)
        pltpu.async_copy(t2.at[i2_b.at[pl.ds(off, C)]], r2, s2)

    def wait_gather(set_i):
        r1, r2, s1, s2 = rows[set_i]
        pltpu.make_async_copy(t1.at[i1_b.at[pl.ds(0, C)]], r1, s1).wait()
        pltpu.make_async_copy(t2.at[i2_b.at[pl.ds(0, C)]], r2, s2).wait()

    def add_and_store(cl, blk, set_i):
        r1, r2, _, _ = rows[set_i]
        ob, osem = ost[set_i]
        gc = blk * CB + cl

        @pl.when(gc >= 2)
        def _():
            pltpu.make_async_copy(ob, out.at[pl.ds(0, C)], osem).wait()

        def addbody(t, carry2):
            for u in range(D // 32):
                sl = pl.ds(u * 16, 16)
                a = plsc.bitcast(r1[t, sl], jnp.bfloat16)
                b = plsc.bitcast(r2[t, sl], jnp.bfloat16)
                lo, hi = plsc.unpack(a + b, format=plsc.PackFormat.INTERLEAVED)
                ob[t, pl.ds(u * 32, 16)] = lo
                ob[t, pl.ds(u * 32 + 16, 16)] = hi
            return carry2

        lax.fori_loop(0, C, addbody, 0)
        base = base0 + blk * IB + cl * C
        pltpu.async_copy(ob, out.at[pl.ds(base, C)], osem)

    def block(blk, carry):
        bbase = base0 + blk * IB
        cps = [pltpu.async_copy(src.at[pl.ds(bbase, IB)], dst, semIdx)
               for src, dst in zip((cc, nb, il, inum, ip, iw),
                                   (cc_v, nb_v, il_v, in_v, ip_v, iw_v))]
        for cp in cps:
            cp.wait()

        def ixbody(j, carry2):
            sl = pl.ds(j * 16, 16)
            i1_b[sl] = lax.rem(cc_v[sl], 300)
            i2_b[sl] = nb_v[sl] * 16 + il_v[sl] * 8 + in_v[sl] * 4 \
                + ip_v[sl] * 2 + iw_v[sl]
            return carry2

        lax.fori_loop(0, IB // 16, ixbody, 0)
        issue_gather(0, 0)

        def pair(h, carry2):
            c0 = 2 * h
            issue_gather(c0 + 1, 1)
            wait_gather(0)
            add_and_store(c0, blk, 0)

            @pl.when(h < HB - 1)
            def _():
                issue_gather(c0 + 2, 0)

            wait_gather(1)
            add_and_store(c0 + 1, blk, 1)
            return carry2

        lax.fori_loop(0, HB, pair, 0)
        return carry

    lax.fori_loop(0, NBLK, block, 0)
    pltpu.make_async_copy(oS0, out.at[pl.ds(0, C)], sO0).wait()
    pltpu.make_async_copy(oS1, out.at[pl.ds(0, C)], sO1).wait()


_sc_kernel = functools.partial(
    pl.kernel,
    mesh=plsc.VectorSubcoreMesh(core_axis_name="c", subcore_axis_name="s"),
    out_type=jax.ShapeDtypeStruct((N, D), jnp.float32),
    compiler_params=pltpu.CompilerParams(needs_layout_passes=False,
                                         use_tc_tiling_on_sc=False),
    scratch_types=[
        pltpu.VMEM((IB,), jnp.int32),
        pltpu.VMEM((IB,), jnp.int32),
        pltpu.VMEM((IB,), jnp.int32),
        pltpu.VMEM((IB,), jnp.int32),
        pltpu.VMEM((IB,), jnp.int32),
        pltpu.VMEM((IB,), jnp.int32),
        pltpu.VMEM((IB,), jnp.int32),
        pltpu.VMEM((IB,), jnp.int32),
        pltpu.VMEM((C, D // 2), jnp.int32),
        pltpu.VMEM((C, D // 2), jnp.int32),
        pltpu.VMEM((C, D // 2), jnp.int32),
        pltpu.VMEM((C, D // 2), jnp.int32),
        pltpu.VMEM((C, D), jnp.float32),
        pltpu.VMEM((C, D), jnp.float32),
        pltpu.SemaphoreType.DMA,
        pltpu.SemaphoreType.DMA,
        pltpu.SemaphoreType.DMA,
        pltpu.SemaphoreType.DMA,
        pltpu.SemaphoreType.DMA,
        pltpu.SemaphoreType.DMA,
        pltpu.SemaphoreType.DMA,
    ],
)(_sc_body)


def kernel(char_code, num_bytes, is_letter, is_number, is_punctuation,
           is_whitespace, c_table, n_table, l_table, num_table, p_table,
           w_table, dense_kernel, dense_bias):
    t1, t2 = _prep_tables(c_table, n_table, l_table, num_table, p_table,
                          w_table, dense_kernel, dense_bias)
    perm = jnp.array([32 * k + (16 * (i % 2) + i // 2)
                      for k in range(D // 32) for i in range(32)],
                     dtype=jnp.int32)
    t1 = lax.bitcast_convert_type(
        t1[:, perm].astype(jnp.bfloat16).reshape(300, D // 2, 2), jnp.int32)
    t2 = lax.bitcast_convert_type(
        t2[:, perm].astype(jnp.bfloat16).reshape(80, D // 2, 2), jnp.int32)
    cc = char_code.reshape(N).astype(jnp.int32)
    nb = num_bytes.reshape(N).astype(jnp.int32)
    il = is_letter.reshape(N).astype(jnp.int32)
    inum = is_number.reshape(N).astype(jnp.int32)
    ip = is_punctuation.reshape(N).astype(jnp.int32)
    iw = is_whitespace.reshape(N).astype(jnp.int32)
    out = _sc_kernel(cc, nb, il, inum, ip, iw, t1, t2)
    return out.reshape(B, P, H, W, D)


# speedup vs baseline: 7.5408x; 3.5219x over previous
import functools

import jax
import jax.numpy as jnp
from jax import lax
from jax.experimental import pallas as pl
from jax.experimental.pallas import tpu as pltpu
from jax.experimental.pallas import tpu_sc as plsc

B, P, H, W = 16, 4, 64, 128
N = B * P * H * W
D = 256
NC, NS = 2, 16
NW = NC * NS
NT = N // NW
C = 128
IB = 2048
CB = IB // C
HB = CB // 2
NBLK = NT // IB
R2 = 80
R1 = 300


def _prep_body(c_ref, n_ref, l_ref, num_ref, p_ref, w_ref, dk_ref, b_ref,
               t1_ref, t2_ref):
    dk = dk_ref[...]
    t1_ref[...] = jnp.dot(c_ref[...], dk[0:64, :],
                          preferred_element_type=jnp.float32)
    n_proj = jnp.dot(n_ref[...], dk[64:80, :],
                     preferred_element_type=jnp.float32)
    l_proj = jnp.dot(l_ref[...], dk[80:112, :],
                     preferred_element_type=jnp.float32)
    num_proj = jnp.dot(num_ref[...], dk[112:144, :],
                       preferred_element_type=jnp.float32)
    p_proj = jnp.dot(p_ref[...], dk[144:176, :],
                     preferred_element_type=jnp.float32)
    w_proj = jnp.dot(w_ref[...], dk[176:208, :],
                     preferred_element_type=jnp.float32)

    idx = lax.broadcasted_iota(jnp.int32, (R2, 1), 0)
    nb = idx // 16
    lbit = (idx // 8) % 2
    nbit = (idx // 4) % 2
    pbit = (idx // 2) % 2
    wbit = idx % 2

    acc = b_ref[...]
    for k in range(5):
        acc = acc + jnp.where(nb == k, 1.0, 0.0) * n_proj[k:k + 1, :]
    acc = acc + jnp.where(lbit == 1, l_proj[1:2, :], l_proj[0:1, :])
    acc = acc + jnp.where(nbit == 1, num_proj[1:2, :], num_proj[0:1, :])
    acc = acc + jnp.where(pbit == 1, p_proj[1:2, :], p_proj[0:1, :])
    acc = acc + jnp.where(wbit == 1, w_proj[1:2, :], w_proj[0:1, :])
    t2_ref[...] = acc


def _prep_tables(c_table, n_table, l_table, num_table, p_table, w_table,
                 dense_kernel, dense_bias):
    return pl.pallas_call(
        _prep_body,
        out_shape=[
            jax.ShapeDtypeStruct((R1, D), jnp.float32),
            jax.ShapeDtypeStruct((R2, D), jnp.float32),
        ],
    )(c_table, n_table, l_table, num_table, p_table, w_table,
      dense_kernel, dense_bias.reshape(1, D))


def _sum_body(t1_ref, t2_ref, t12_ref):
    t12_ref[...] = t2_ref[...][:, None, :] + t1_ref[...][None, :, :]


def _sum_tables(t1, t2):
    return pl.pallas_call(
        _sum_body,
        grid=(R2 // 8,),
        in_specs=[
            pl.BlockSpec((R1, D), lambda j: (0, 0)),
            pl.BlockSpec((8, D), lambda j: (j, 0)),
        ],
        out_specs=pl.BlockSpec((8, R1, D), lambda j: (j, 0, 0)),
        out_shape=jax.ShapeDtypeStruct((R2, R1, D), jnp.float32),
    )(t1, t2)


def _sc_body(cc, nb, il, inum, ip, iw, t12, out,
             cc_v, nb_v, il_v, in_v, ip_v, iw_v, i_b,
             bufA, bufB, semIdx, gA, gB, oA, oB):
    wid = lax.axis_index("s") * NC + lax.axis_index("c")
    base0 = wid * NT

    def issue_gather(cl, buf, gsem):
        pltpu.async_copy(t12.at[i_b.at[pl.ds(cl * C, C)]], buf, gsem)

    def wait_gather(buf, gsem):
        pltpu.make_async_copy(t12.at[i_b.at[pl.ds(0, C)]], buf, gsem).wait()

    def writeout(cl, blk, buf, osem):
        base = base0 + blk * IB + cl * C
        pltpu.async_copy(buf, out.at[pl.ds(base, C)], osem)

    def wait_out(buf, osem):
        pltpu.make_async_copy(buf, out.at[pl.ds(0, C)], osem).wait()

    def block(blk, carry):
        bbase = base0 + blk * IB
        cps = [pltpu.async_copy(src.at[pl.ds(bbase, IB)], dst, semIdx)
               for src, dst in zip((cc, nb, il, inum, ip, iw),
                                   (cc_v, nb_v, il_v, in_v, ip_v, iw_v))]
        for cp in cps:
            cp.wait()

        def ixbody(j, carry2):
            sl = pl.ds(j * 16, 16)
            i_b[sl] = (nb_v[sl] * 16 + il_v[sl] * 8 + in_v[sl] * 4
                       + ip_v[sl] * 2 + iw_v[sl]) * R1 + lax.rem(cc_v[sl], R1)
            return carry2

        lax.fori_loop(0, IB // 16, ixbody, 0)

        @pl.when(blk > 0)
        def _():
            wait_out(bufA, oA)
            wait_out(bufB, oB)

        issue_gather(0, bufA, gA)
        issue_gather(1, bufB, gB)

        def pair(h, carry2):
            c0 = 2 * h
            wait_gather(bufA, gA)
            writeout(c0, blk, bufA, oA)
            wait_gather(bufB, gB)
            writeout(c0 + 1, blk, bufB, oB)

            @pl.when(h < HB - 1)
            def _():
                wait_out(bufA, oA)
                issue_gather(c0 + 2, bufA, gA)
                wait_out(bufB, oB)
                issue_gather(c0 + 3, bufB, gB)

            return carry2

        lax.fori_loop(0, HB, pair, 0)
        return carry

    lax.fori_loop(0, NBLK, block, 0)
    wait_out(bufA, oA)
    wait_out(bufB, oB)


_sc_kernel = functools.partial(
    pl.kernel,
    mesh=plsc.VectorSubcoreMesh(core_axis_name="c", subcore_axis_name="s"),
    out_type=jax.ShapeDtypeStruct((N, D), jnp.float32),
    scratch_types=[
        pltpu.VMEM((IB,), jnp.int32),
        pltpu.VMEM((IB,), jnp.int32),
        pltpu.VMEM((IB,), jnp.int32),
        pltpu.VMEM((IB,), jnp.int32),
        pltpu.VMEM((IB,), jnp.int32),
        pltpu.VMEM((IB,), jnp.int32),
        pltpu.VMEM((IB,), jnp.int32),
        pltpu.VMEM((C, D), jnp.float32),
        pltpu.VMEM((C, D), jnp.float32),
        pltpu.SemaphoreType.DMA,
        pltpu.SemaphoreType.DMA,
        pltpu.SemaphoreType.DMA,
        pltpu.SemaphoreType.DMA,
        pltpu.SemaphoreType.DMA,
    ],
)(_sc_body)


def kernel(char_code, num_bytes, is_letter, is_number, is_punctuation,
           is_whitespace, c_table, n_table, l_table, num_table, p_table,
           w_table, dense_kernel, dense_bias):
    t1, t2 = _prep_tables(c_table, n_table, l_table, num_table, p_table,
                          w_table, dense_kernel, dense_bias)
    t12 = _sum_tables(t1, t2).reshape(R2 * R1, D)
    cc = char_code.reshape(N).astype(jnp.int32)
    nb = num_bytes.reshape(N).astype(jnp.int32)
    il = is_letter.reshape(N).astype(jnp.int32)
    inum = is_number.reshape(N).astype(jnp.int32)
    ip = is_punctuation.reshape(N).astype(jnp.int32)
    iw = is_whitespace.reshape(N).astype(jnp.int32)
    out = _sc_kernel(cc, nb, il, inum, ip, iw, t12)
    return out.reshape(B, P, H, W, D)
